# initial kernel scaffold (unmeasured)
import jax
import jax.numpy as jnp
from jax import lax
from jax.experimental import pallas as pl
from jax.experimental.pallas import tpu as pltpu


def kernel(
    x,
):
    def body(*refs):
        pass

    out_shape = jax.ShapeDtypeStruct(..., jnp.float32)
    return pl.pallas_call(body, out_shape=out_shape)(...)


# baseline (device time: 146938 ns/iter reference)
import jax
import jax.numpy as jnp
from jax import lax
from jax.experimental import pallas as pl
from jax.experimental.pallas import tpu as pltpu

N_DEV = 16
ROWS = 1024
COLS = 1024
CHUNK = ROWS // N_DEV


def kernel(x):
    def body(
        x_ref,
        out_ref,
        stage,
        rs_recv,
        rs_send_sems,
        rs_recv_sems,
        ag_send_sems,
        ag_recv_sems,
    ):
        my = lax.axis_index("i")
        left = lax.rem(my - 1 + N_DEV, N_DEV)
        right = lax.rem(my + 1, N_DEV)

        barrier_sem = pltpu.get_barrier_semaphore()
        for nbr in (left, right):
            pl.semaphore_signal(
                barrier_sem,
                inc=1,
                device_id=(nbr,),
                device_id_type=pl.DeviceIdType.MESH,
            )
        pl.semaphore_wait(barrier_sem, 2)

        def rows(c):
            return pl.ds(c * CHUNK, CHUNK)

        stage[0, :, :] = x_ref[0, rows(my), :]
        for h in range(N_DEV - 1):
            s = h % 2
            rdma = pltpu.make_async_remote_copy(
                src_ref=stage.at[s],
                dst_ref=rs_recv.at[h],
                send_sem=rs_send_sems.at[s],
                recv_sem=rs_recv_sems.at[h],
                device_id=(right,),
                device_id_type=pl.DeviceIdType.MESH,
            )
            rdma.start()
            rdma.wait()
            c = lax.rem(my - h - 1 + N_DEV, N_DEV)
            if h < N_DEV - 2:
                stage[(h + 1) % 2, :, :] = rs_recv[h] + x_ref[0, rows(c), :]
            else:
                out_ref[rows(c), :] = rs_recv[h] + x_ref[0, rows(c), :]

        for h in range(N_DEV - 1):
            c_send = lax.rem(my + 1 - h + N_DEV, N_DEV)
            rdma = pltpu.make_async_remote_copy(
                src_ref=out_ref.at[rows(c_send)],
                dst_ref=out_ref.at[rows(c_send)],
                send_sem=ag_send_sems.at[h],
                recv_sem=ag_recv_sems.at[h],
                device_id=(right,),
                device_id_type=pl.DeviceIdType.MESH,
            )
            rdma.start()
            rdma.wait()

    return pl.pallas_call(
        body,
        out_shape=jax.ShapeDtypeStruct((ROWS, COLS), jnp.float32),
        in_specs=[pl.BlockSpec(memory_space=pltpu.VMEM)],
        out_specs=pl.BlockSpec(memory_space=pltpu.VMEM),
        scratch_shapes=[
            pltpu.VMEM((2, CHUNK, COLS), jnp.float32),
            pltpu.VMEM((N_DEV - 1, CHUNK, COLS), jnp.float32),
            pltpu.SemaphoreType.DMA((2,)),
            pltpu.SemaphoreType.DMA((N_DEV - 1,)),
            pltpu.SemaphoreType.DMA((N_DEV - 1,)),
            pltpu.SemaphoreType.DMA((N_DEV - 1,)),
        ],
        compiler_params=pltpu.CompilerParams(collective_id=0),
    )(x)


# device time: 82642 ns/iter; 1.7780x vs baseline; 1.7780x over previous
import jax
import jax.numpy as jnp
from jax import lax
from jax.experimental import pallas as pl
from jax.experimental.pallas import tpu as pltpu

N_DEV = 16
ROWS = 1024
COLS = 1024
BLOCK = ROWS // 4
HALF = BLOCK // 2
ZC = HALF // 4


def kernel(x):
    def body(
        x_ref,
        out_ref,
        stage_a,
        recv_a,
        stage_b,
        recv_b,
        send_a_sems,
        recv_a_sems,
        send_b_sems,
        recv_b_sems,
        agb_send_sems,
        agb_recv_sems,
        agc_send_sems,
        agc_recv_sems,
    ):
        my = lax.axis_index("i")
        p = my // 4
        j = lax.rem(my, 4)

        fwd = p * 4 + lax.rem(j + 1, 4)
        bwd = p * 4 + lax.rem(j + 3, 4)
        fz = lax.rem(p + 1, 4) * 4 + j
        bz = lax.rem(p + 3, 4) * 4 + j

        barrier_sem = pltpu.get_barrier_semaphore()
        for nbr in (fwd, bwd, fz, bz):
            pl.semaphore_signal(
                barrier_sem,
                inc=1,
                device_id=(nbr,),
                device_id_type=pl.DeviceIdType.MESH,
            )
        pl.semaphore_wait(barrier_sem, 4)

        plane_tgt = (fwd, bwd)
        z_tgt = (fz, bz)

        for d in (0, 1):
            stage_a[d, 0, :, :] = x_ref[0, pl.ds(j * BLOCK + d * HALF, HALF), :]
        for h in range(3):
            rdmas = []
            for d in (0, 1):
                rdma = pltpu.make_async_remote_copy(
                    src_ref=stage_a.at[d, h % 2],
                    dst_ref=recv_a.at[d, h],
                    send_sem=send_a_sems.at[d, h % 2],
                    recv_sem=recv_a_sems.at[d, h],
                    device_id=(plane_tgt[d],),
                    device_id_type=pl.DeviceIdType.MESH,
                )
                rdma.start()
                rdmas.append(rdma)
            for rdma in rdmas:
                rdma.wait()
            for d in (0, 1):
                c = lax.rem(j - h - 1 + 8, 4) if d == 0 else lax.rem(j + h + 1, 4)
                rows = pl.ds(c * BLOCK + d * HALF, HALF)
                if h < 2:
                    stage_a[d, (h + 1) % 2, :, :] = recv_a[d, h] + x_ref[0, rows, :]
                else:
                    out_ref[rows, :] = recv_a[d, h] + x_ref[0, rows, :]

        c1 = lax.rem(j + 1, 4)
        c2 = lax.rem(j + 3, 4)
        rbase = (c1 * BLOCK, c2 * BLOCK + HALF)

        for d in (0, 1):
            stage_b[d, 0, :, :] = out_ref[pl.ds(rbase[d] + p * ZC, ZC), :]
        for h in range(3):
            rdmas = []
            for d in (0, 1):
                rdma = pltpu.make_async_remote_copy(
                    src_ref=stage_b.at[d, h % 2],
                    dst_ref=recv_b.at[d, h],
                    send_sem=send_b_sems.at[d, h % 2],
                    recv_sem=recv_b_sems.at[d, h],
                    device_id=(z_tgt[d],),
                    device_id_type=pl.DeviceIdType.MESH,
                )
                rdma.start()
                rdmas.append(rdma)
            for rdma in rdmas:
                rdma.wait()
            for d in (0, 1):
                q = lax.rem(p - h - 1 + 8, 4) if d == 0 else lax.rem(p + h + 1, 4)
                rows = pl.ds(rbase[d] + q * ZC, ZC)
                if h < 2:
                    stage_b[d, (h + 1) % 2, :, :] = recv_b[d, h] + out_ref[rows, :]
                else:
                    out_ref[rows, :] = recv_b[d, h] + out_ref[rows, :]

        for h in range(3):
            rdmas = []
            for d in (0, 1):
                q = lax.rem(p + 1 - h + 8, 4) if d == 0 else lax.rem(p + 3 + h, 4)
                rows = pl.ds(rbase[d] + q * ZC, ZC)
                rdma = pltpu.make_async_remote_copy(
                    src_ref=out_ref.at[rows],
                    dst_ref=out_ref.at[rows],
                    send_sem=agb_send_sems.at[d, h],
                    recv_sem=agb_recv_sems.at[d, h],
                    device_id=(z_tgt[d],),
                    device_id_type=pl.DeviceIdType.MESH,
                )
                rdma.start()
                rdmas.append(rdma)
            for rdma in rdmas:
                rdma.wait()

        for h in range(3):
            rdmas = []
            for d in (0, 1):
                b = lax.rem(j + 1 - h + 8, 4) if d == 0 else lax.rem(j + 3 + h, 4)
                rows = pl.ds(b * BLOCK + d * HALF, HALF)
                rdma = pltpu.make_async_remote_copy(
                    src_ref=out_ref.at[rows],
                    dst_ref=out_ref.at[rows],
                    send_sem=agc_send_sems.at[d, h],
                    recv_sem=agc_recv_sems.at[d, h],
                    device_id=(plane_tgt[d],),
                    device_id_type=pl.DeviceIdType.MESH,
                )
                rdma.start()
                rdmas.append(rdma)
            for rdma in rdmas:
                rdma.wait()

    return pl.pallas_call(
        body,
        out_shape=jax.ShapeDtypeStruct((ROWS, COLS), jnp.float32),
        in_specs=[pl.BlockSpec(memory_space=pltpu.VMEM)],
        out_specs=pl.BlockSpec(memory_space=pltpu.VMEM),
        scratch_shapes=[
            pltpu.VMEM((2, 2, HALF, COLS), jnp.float32),
            pltpu.VMEM((2, 3, HALF, COLS), jnp.float32),
            pltpu.VMEM((2, 2, ZC, COLS), jnp.float32),
            pltpu.VMEM((2, 3, ZC, COLS), jnp.float32),
            pltpu.SemaphoreType.DMA((2, 2)),
            pltpu.SemaphoreType.DMA((2, 3)),
            pltpu.SemaphoreType.DMA((2, 2)),
            pltpu.SemaphoreType.DMA((2, 3)),
            pltpu.SemaphoreType.DMA((2, 3)),
            pltpu.SemaphoreType.DMA((2, 3)),
            pltpu.SemaphoreType.DMA((2, 3)),
            pltpu.SemaphoreType.DMA((2, 3)),
        ],
        compiler_params=pltpu.CompilerParams(collective_id=0),
    )(x)


# device time: 82178 ns/iter; 1.7880x vs baseline; 1.0056x over previous
import jax
import jax.numpy as jnp
from jax import lax
from jax.experimental import pallas as pl
from jax.experimental.pallas import tpu as pltpu

N_DEV = 16
ROWS = 1024
COLS = 1024
BLOCK = ROWS // 4
HALF = BLOCK // 2
ZC = HALF // 4
SC = COLS // 2
N_HOPS = 12


def kernel(x):
    def body(
        x_ref,
        out_ref,
        stage_a,
        recv_a,
        stage_b,
        recv_b,
        sa_send,
        sa_recv,
        sb_send,
        sb_recv,
        agb_send,
        agb_recv,
        agc_send,
        agc_recv,
    ):
        my = lax.axis_index("i")
        p = my // 4
        j = lax.rem(my, 4)

        fwd = p * 4 + lax.rem(j + 1, 4)
        bwd = p * 4 + lax.rem(j + 3, 4)
        fz = lax.rem(p + 1, 4) * 4 + j
        bz = lax.rem(p + 3, 4) * 4 + j
        plane_tgt = (fwd, bwd)
        z_tgt = (fz, bz)

        barrier_sem = pltpu.get_barrier_semaphore()
        for nbr in (fwd, bwd, fz, bz):
            pl.semaphore_signal(
                barrier_sem,
                inc=1,
                device_id=(nbr,),
                device_id_type=pl.DeviceIdType.MESH,
            )
        pl.semaphore_wait(barrier_sem, 4)

        c1 = lax.rem(j + 1, 4)
        c2 = lax.rem(j + 3, 4)
        rbase = (c1 * BLOCK, c2 * BLOCK + HALF)

        def cols(st):
            return pl.ds(st * SC, SC)

        def remote_copy(src, dst, send_sem, recv_sem, tgt):
            return pltpu.make_async_remote_copy(
                src_ref=src,
                dst_ref=dst,
                send_sem=send_sem,
                recv_sem=recv_sem,
                device_id=(tgt,),
                device_id_type=pl.DeviceIdType.MESH,
            )

        def issue(st, h):
            rdmas = []
            if h < 3:
                if h == 0:
                    for d in (0, 1):
                        stage_a[st, d, 0, :, :] = x_ref[
                            0, pl.ds(j * BLOCK + d * HALF, HALF), cols(st)
                        ]
                for d in (0, 1):
                    r = remote_copy(
                        stage_a.at[st, d, h],
                        recv_a.at[st, d, h],
                        sa_send.at[st, d, h],
                        sa_recv.at[st, d, h],
                        plane_tgt[d],
                    )
                    r.start()
                    rdmas.append((d, r))
            elif h < 6:
                hh = h - 3
                if hh == 0:
                    for d in (0, 1):
                        stage_b[st, d, 0, :, :] = out_ref[
                            pl.ds(rbase[d] + p * ZC, ZC), cols(st)
                        ]
                for d in (0, 1):
                    r = remote_copy(
                        stage_b.at[st, d, hh],
                        recv_b.at[st, d, hh],
                        sb_send.at[st, d, hh],
                        sb_recv.at[st, d, hh],
                        z_tgt[d],
                    )
                    r.start()
                    rdmas.append((d, r))
            elif h < 9:
                hh = h - 6
                for d in (0, 1):
                    q = lax.rem(p + 1 - hh + 8, 4) if d == 0 else lax.rem(p + 3 + hh, 4)
                    ref = out_ref.at[pl.ds(rbase[d] + q * ZC, ZC), cols(st)]
                    r = remote_copy(
                        ref,
                        ref,
                        agb_send.at[st, d, hh],
                        agb_recv.at[st, d, hh],
                        z_tgt[d],
                    )
                    r.start()
                    rdmas.append((d, r))
            else:
                hh = h - 9
                for d in (0, 1):
                    b = lax.rem(j + 1 - hh + 8, 4) if d == 0 else lax.rem(j + 3 + hh, 4)
                    ref = out_ref.at[pl.ds(b * BLOCK + d * HALF, HALF), cols(st)]
                    r = remote_copy(
                        ref,
                        ref,
                        agc_send.at[st, d, hh],
                        agc_recv.at[st, d, hh],
                        plane_tgt[d],
                    )
                    r.start()
                    rdmas.append((d, r))
            return rdmas

        def finish(st, h, rdmas):
            if h < 3:
                for d, r in rdmas:
                    r.wait_recv()
                    c = lax.rem(j - h - 1 + 8, 4) if d == 0 else lax.rem(j + h + 1, 4)
                    rows = pl.ds(c * BLOCK + d * HALF, HALF)
                    if h < 2:
                        stage_a[st, d, h + 1, :, :] = (
                            recv_a[st, d, h] + x_ref[0, rows, cols(st)]
                        )
                    else:
                        out_ref[rows, cols(st)] = (
                            recv_a[st, d, h] + x_ref[0, rows, cols(st)]
                        )
            elif h < 6:
                hh = h - 3
                for d, r in rdmas:
                    r.wait_recv()
                    q = lax.rem(p - hh - 1 + 8, 4) if d == 0 else lax.rem(p + hh + 1, 4)
                    rows = pl.ds(rbase[d] + q * ZC, ZC)
                    if hh < 2:
                        stage_b[st, d, hh + 1, :, :] = (
                            recv_b[st, d, hh] + out_ref[rows, cols(st)]
                        )
                    else:
                        out_ref[rows, cols(st)] = (
                            recv_b[st, d, hh] + out_ref[rows, cols(st)]
                        )
            else:
                for _, r in rdmas:
                    r.wait_recv()

        all_rdmas = []
        pending = {}
        for s in range(N_HOPS + 1):
            for st in (0, 1):
                h = s - st
                if 0 <= h < N_HOPS:
                    pending[st] = issue(st, h)
                    all_rdmas.extend(r for _, r in pending[st])
            for st in (0, 1):
                h = s - st
                if 0 <= h < N_HOPS:
                    finish(st, h, pending[st])

        for r in all_rdmas:
            r.wait_send()

    return pl.pallas_call(
        body,
        out_shape=jax.ShapeDtypeStruct((ROWS, COLS), jnp.float32),
        in_specs=[pl.BlockSpec(memory_space=pltpu.VMEM)],
        out_specs=pl.BlockSpec(memory_space=pltpu.VMEM),
        scratch_shapes=[
            pltpu.VMEM((2, 2, 3, HALF, SC), jnp.float32),
            pltpu.VMEM((2, 2, 3, HALF, SC), jnp.float32),
            pltpu.VMEM((2, 2, 3, ZC, SC), jnp.float32),
            pltpu.VMEM((2, 2, 3, ZC, SC), jnp.float32),
            pltpu.SemaphoreType.DMA((2, 2, 3)),
            pltpu.SemaphoreType.DMA((2, 2, 3)),
            pltpu.SemaphoreType.DMA((2, 2, 3)),
            pltpu.SemaphoreType.DMA((2, 2, 3)),
            pltpu.SemaphoreType.DMA((2, 2, 3)),
            pltpu.SemaphoreType.DMA((2, 2, 3)),
            pltpu.SemaphoreType.DMA((2, 2, 3)),
            pltpu.SemaphoreType.DMA((2, 2, 3)),
        ],
        compiler_params=pltpu.CompilerParams(collective_id=0),
    )(x)
